# baseline (speedup 1.0000x reference)
import jax
import jax.numpy as jnp
from jax.experimental import pallas as pl
from jax.experimental.pallas import tpu as pltpu


def _mask_body(k_ref, x_ref, o_ref, bits_ref):
    x = x_ref[...]
    bits_ref[...] = jax.lax.bitcast_convert_type(jnp.abs(x), jnp.int32)
    k = k_ref[0]
    rows = x.shape[0]

    n = x.shape[1]
    chunks = 16
    w = n // chunks

    def body(i, prefix):
        cand = prefix | (jnp.int32(1) << (jnp.int32(30) - i))
        cnt = jnp.zeros((rows, 1), jnp.int32)
        parts = [
            jnp.sum((bits_ref[:, c * w:(c + 1) * w] >= cand).astype(jnp.int32),
                    axis=1, keepdims=True)
            for c in range(chunks)
        ]
        for p in parts:
            cnt = cnt + p
        return jnp.where(cnt >= k, cand, prefix)

    prefix = jax.lax.fori_loop(0, 31, body, jnp.zeros((rows, 1), jnp.int32))
    o_ref[...] = jnp.where(bits_ref[...] >= prefix, x, 0.0)


def kernel(scores, k):
    b, n = scores.shape
    rows_per_block = 8
    k_arr = jnp.reshape(jnp.asarray(k, jnp.int32), (1,))
    return pl.pallas_call(
        _mask_body,
        grid=(b // rows_per_block,),
        in_specs=[
            pl.BlockSpec(memory_space=pltpu.SMEM),
            pl.BlockSpec((rows_per_block, n), lambda i: (i, 0)),
        ],
        out_specs=pl.BlockSpec((rows_per_block, n), lambda i: (i, 0)),
        out_shape=jax.ShapeDtypeStruct((b, n), scores.dtype),
        scratch_shapes=[pltpu.VMEM((rows_per_block, n), jnp.int32)],
    )(k_arr, scores)


# speedup vs baseline: 1.2769x; 1.2769x over previous
import jax
import jax.numpy as jnp
from jax.experimental import pallas as pl
from jax.experimental.pallas import tpu as pltpu


def _mask_body(k_ref, x_ref, o_ref, bits_ref):
    x = x_ref[...]
    bits_ref[...] = jax.lax.bitcast_convert_type(jnp.abs(x), jnp.int32)
    k = k_ref[0]
    rows = x.shape[0]

    n = x.shape[1]
    chunks = 8
    w = n // chunks

    def body(i, prefix):
        cand = prefix | (jnp.int32(1) << (jnp.int32(30) - i))
        cnt = jnp.zeros((rows, 1), jnp.int32)
        parts = [
            jnp.sum((bits_ref[:, c * w:(c + 1) * w] >= cand).astype(jnp.int32),
                    axis=1, keepdims=True)
            for c in range(chunks)
        ]
        for p in parts:
            cnt = cnt + p
        return jnp.where(cnt >= k, cand, prefix)

    prefix = jax.lax.fori_loop(0, 31, body, jnp.zeros((rows, 1), jnp.int32))
    o_ref[...] = jnp.where(bits_ref[...] >= prefix, x, 0.0)


def kernel(scores, k):
    b, n = scores.shape
    rows_per_block = 16
    k_arr = jnp.reshape(jnp.asarray(k, jnp.int32), (1,))
    return pl.pallas_call(
        _mask_body,
        grid=(b // rows_per_block,),
        in_specs=[
            pl.BlockSpec(memory_space=pltpu.SMEM),
            pl.BlockSpec((rows_per_block, n), lambda i: (i, 0)),
        ],
        out_specs=pl.BlockSpec((rows_per_block, n), lambda i: (i, 0)),
        out_shape=jax.ShapeDtypeStruct((b, n), scores.dtype),
        scratch_shapes=[pltpu.VMEM((rows_per_block, n), jnp.int32)],
    )(k_arr, scores)


# speedup vs baseline: 1.4133x; 1.1068x over previous
import jax
import jax.numpy as jnp
from jax.experimental import pallas as pl
from jax.experimental.pallas import tpu as pltpu


def _mask_body(k_ref, x_ref, o_ref, bits_ref):
    x = x_ref[...]
    bits_ref[...] = jax.lax.bitcast_convert_type(jnp.abs(x), jnp.int32)
    k = k_ref[0]
    rows = x.shape[0]

    n = x.shape[1]
    chunks = 8
    w = n // chunks

    def body(i, prefix):
        cand = prefix | (jnp.int32(1) << (jnp.int32(30) - i))
        cnt = jnp.zeros((rows, 1), jnp.int32)
        parts = [
            jnp.sum((bits_ref[:, c * w:(c + 1) * w] >= cand).astype(jnp.int32),
                    axis=1, keepdims=True)
            for c in range(chunks)
        ]
        for p in parts:
            cnt = cnt + p
        return jnp.where(cnt >= k, cand, prefix)

    prefix = jax.lax.fori_loop(0, 31, body, jnp.zeros((rows, 1), jnp.int32))
    o_ref[...] = jnp.where(bits_ref[...] >= prefix, x, 0.0)


def kernel(scores, k):
    b, n = scores.shape
    rows_per_block = 32
    k_arr = jnp.reshape(jnp.asarray(k, jnp.int32), (1,))
    return pl.pallas_call(
        _mask_body,
        grid=(b // rows_per_block,),
        in_specs=[
            pl.BlockSpec(memory_space=pltpu.SMEM),
            pl.BlockSpec((rows_per_block, n), lambda i: (i, 0)),
        ],
        out_specs=pl.BlockSpec((rows_per_block, n), lambda i: (i, 0)),
        out_shape=jax.ShapeDtypeStruct((b, n), scores.dtype),
        scratch_shapes=[pltpu.VMEM((rows_per_block, n), jnp.int32)],
    )(k_arr, scores)


# speedup vs baseline: 1.4694x; 1.0397x over previous
import jax
import jax.numpy as jnp
from jax.experimental import pallas as pl
from jax.experimental.pallas import tpu as pltpu


def _mask_body(k_ref, x_ref, o_ref, bits_ref):
    x = x_ref[...]
    bits_ref[...] = jax.lax.bitcast_convert_type(jnp.abs(x), jnp.int32)
    k = k_ref[0]
    rows = x.shape[0]

    n = x.shape[1]
    chunks = 8
    w = n // chunks

    def body(i, prefix):
        cand = prefix | (jnp.int32(1) << (jnp.int32(30) - i))
        cnt = jnp.zeros((rows, 1), jnp.int32)
        parts = [
            jnp.sum((bits_ref[:, c * w:(c + 1) * w] >= cand).astype(jnp.int32),
                    axis=1, keepdims=True)
            for c in range(chunks)
        ]
        for p in parts:
            cnt = cnt + p
        return jnp.where(cnt >= k, cand, prefix)

    prefix = jax.lax.fori_loop(0, 31, body, jnp.zeros((rows, 1), jnp.int32))
    o_ref[...] = jnp.where(bits_ref[...] >= prefix, x, 0.0)


def kernel(scores, k):
    b, n = scores.shape
    rows_per_block = 64
    k_arr = jnp.reshape(jnp.asarray(k, jnp.int32), (1,))
    return pl.pallas_call(
        _mask_body,
        grid=(b // rows_per_block,),
        in_specs=[
            pl.BlockSpec(memory_space=pltpu.SMEM),
            pl.BlockSpec((rows_per_block, n), lambda i: (i, 0)),
        ],
        out_specs=pl.BlockSpec((rows_per_block, n), lambda i: (i, 0)),
        out_shape=jax.ShapeDtypeStruct((b, n), scores.dtype),
        scratch_shapes=[pltpu.VMEM((rows_per_block, n), jnp.int32)],
    )(k_arr, scores)
